# Initial kernel scaffold; baseline (speedup 1.0000x reference)
import jax
import jax.numpy as jnp
from jax.experimental import pallas as pl


def kernel(x, edge_index, trace_h, W_enc, W_msg, b_msg, w_eh, W_agg, W_self, w_hint, W_out):
    raise NotImplementedError("write your pallas kernel here")


# speedup vs baseline: 2.4444x; 2.4444x over previous
import functools

import jax
import jax.numpy as jnp
from jax import lax
from jax.experimental import pallas as pl
from jax.experimental.pallas import tpu as pltpu
from jax.experimental.pallas import tpu_sc as plsc

NC, NS = 2, 16
NW = NC * NS
L = 16
C = 128


def _make_sc_msg(N_pad, E, H):
    n_chunks = E // C
    n_base, n_extra = n_chunks // NW, n_chunks % NW
    rows_per = N_pad // NS
    zr = 128
    n_zero = rows_per // zr
    HJ = H // L

    mesh = plsc.VectorSubcoreMesh(core_axis_name="c", subcore_axis_name="s")

    @functools.partial(
        pl.kernel,
        out_type=jax.ShapeDtypeStruct((NC, N_pad, H), jnp.float32),
        mesh=mesh,
        compiler_params=pltpu.CompilerParams(needs_layout_passes=False),
        scratch_types=[
            pltpu.VMEM((H,), jnp.float32),
            pltpu.VMEM((C,), jnp.int32),
            pltpu.VMEM((C,), jnp.int32),
            pltpu.VMEM((C,), jnp.float32),
            pltpu.VMEM((C, H), jnp.float32),
            pltpu.VMEM((zr, H), jnp.float32),
            pltpu.VMEM_SHARED((N_pad, H), jnp.float32),
            pltpu.SemaphoreType.DMA,
        ],
    )
    def sc_msg(p_hbm, src_hbm, dst_hbm, tr_hbm, weh_hbm, agg_hbm,
               weh_v, sidx_v, didx_v, tr_v, buf_v, zero_v, agg_sh, sem):
        c = lax.axis_index("c")
        s = lax.axis_index("s")
        wid = c * NS + s

        pltpu.sync_copy(weh_hbm, weh_v)

        def _zrow(i, _):
            for j in range(HJ):
                zero_v[i, pl.ds(j * L, L)] = jnp.zeros((L,), jnp.float32)
            return 0
        lax.fori_loop(0, zr, _zrow, 0)
        for k in range(n_zero):
            pltpu.sync_copy(zero_v, agg_sh.at[pl.ds(s * rows_per + k * zr, zr)])
        plsc.subcore_barrier()

        weh = [weh_v[pl.ds(j * L, L)] for j in range(HJ)]

        def _chunk(i, _):
            base = (i * NW + wid) * C
            pltpu.sync_copy(src_hbm.at[pl.ds(base, C)], sidx_v)
            pltpu.sync_copy(dst_hbm.at[pl.ds(base, C)], didx_v)
            pltpu.sync_copy(tr_hbm.at[pl.ds(base, C)], tr_v)
            pltpu.async_copy(p_hbm.at[sidx_v], buf_v, sem).wait()

            def _grp(g, _):
                trv = tr_v[pl.ds(g * L, L)]
                for i in range(L):
                    t = trv[i]
                    e = g * L + i
                    for j in range(HJ):
                        v = buf_v[e, pl.ds(j * L, L)]
                        buf_v[e, pl.ds(j * L, L)] = jnp.maximum(v + t * weh[j], 0.0)
                return 0
            lax.fori_loop(0, C // L, _grp, 0)

            pltpu.sync_copy(buf_v, agg_sh.at[didx_v], add=True)
            return 0

        n_i = n_base + (wid < n_extra).astype(jnp.int32)
        lax.fori_loop(0, n_i, _chunk, 0)
        plsc.subcore_barrier()

        pltpu.sync_copy(agg_sh.at[pl.ds(s * rows_per, rows_per)],
                        agg_hbm.at[c, pl.ds(s * rows_per, rows_per)])

    return sc_msg


def _make_sc_hint(N, E, H):
    n_chunks = E // C
    n_base, n_extra = n_chunks // NW, n_chunks % NW
    HJ = H // L

    mesh = plsc.VectorSubcoreMesh(core_axis_name="c", subcore_axis_name="s")

    @functools.partial(
        pl.kernel,
        out_type=jax.ShapeDtypeStruct((E,), jnp.float32),
        mesh=mesh,
        compiler_params=pltpu.CompilerParams(needs_layout_passes=False),
        scratch_types=[
            pltpu.VMEM((C,), jnp.int32),
            pltpu.VMEM((C,), jnp.int32),
            pltpu.VMEM((C, H), jnp.float32),
            pltpu.VMEM((C, H), jnp.float32),
            pltpu.VMEM((C,), jnp.float32),
            pltpu.SemaphoreType.DMA,
            pltpu.SemaphoreType.DMA,
        ],
    )
    def sc_hint(h_hbm, hw_hbm, src_hbm, dst_hbm, pred_hbm,
                sidx_v, didx_v, bufa_v, bufb_v, pr_v, sema, semb):
        c = lax.axis_index("c")
        s = lax.axis_index("s")
        wid = c * NS + s

        def _chunk(i, _):
            base = (i * NW + wid) * C
            pltpu.sync_copy(src_hbm.at[pl.ds(base, C)], sidx_v)
            pltpu.sync_copy(dst_hbm.at[pl.ds(base, C)], didx_v)
            cpa = pltpu.async_copy(h_hbm.at[sidx_v], bufa_v, sema)
            cpb = pltpu.async_copy(hw_hbm.at[didx_v], bufb_v, semb)
            cpa.wait()
            cpb.wait()

            lane = lax.iota(jnp.int32, L)

            def _grp(g, _):
                vec = jnp.zeros((L,), jnp.float32)
                for i in range(L):
                    e = g * L + i
                    acc = bufa_v[e, pl.ds(0, L)] * bufb_v[e, pl.ds(0, L)]
                    for j in range(1, HJ):
                        acc = acc + (bufa_v[e, pl.ds(j * L, L)]
                                     * bufb_v[e, pl.ds(j * L, L)])
                    s = jnp.sum(acc)
                    vec = jnp.where(lane == i, s, vec)
                pr_v[pl.ds(g * L, L)] = vec
                return 0
            lax.fori_loop(0, C // L, _grp, 0)

            pltpu.sync_copy(pr_v, pred_hbm.at[pl.ds(base, C)])
            return 0

        n_i = n_base + (wid < n_extra).astype(jnp.int32)
        lax.fori_loop(0, n_i, _chunk, 0)

    return sc_hint


def _tc_specs(R, H, n_w):
    row = pl.BlockSpec((R, H), lambda i: (i, 0))
    full = pl.BlockSpec((H, H), lambda i: (0, 0))
    vec = pl.BlockSpec((1, H), lambda i: (0, 0))
    return row, full, vec


def _make_tc_encode(N, D, H, R):
    row = pl.BlockSpec((R, D), lambda i: (i, 0))
    full = pl.BlockSpec((D, H), lambda i: (0, 0))
    vec = pl.BlockSpec((1, H), lambda i: (0, 0))
    out_row = pl.BlockSpec((R, H), lambda i: (i, 0))

    def body(x_ref, wenc_ref, wmsg_ref, b_ref, xe_ref, p_ref):
        xe = jnp.dot(x_ref[...], wenc_ref[...], preferred_element_type=jnp.float32, precision=lax.Precision.HIGHEST)
        xe_ref[...] = xe
        p_ref[...] = jnp.dot(xe, wmsg_ref[...], preferred_element_type=jnp.float32, precision=lax.Precision.HIGHEST) + b_ref[...]

    return pl.pallas_call(
        body,
        grid=(N // R,),
        in_specs=[row, full, full, vec],
        out_specs=[out_row, out_row],
        out_shape=[jax.ShapeDtypeStruct((N, H), jnp.float32)] * 2,
    )


def _make_tc_update(N, H, R, with_p):
    row = pl.BlockSpec((R, H), lambda i: (i, 0))
    agg_spec = pl.BlockSpec((NC, R, H), lambda i: (0, i, 0))
    full = pl.BlockSpec((H, H), lambda i: (0, 0))
    vec = pl.BlockSpec((1, H), lambda i: (0, 0))

    def body(agg_ref, h_ref, xe_ref, wagg_ref, wself_ref, wmsg_ref, b_ref,
             whint_ref, hn_ref, hw_ref, *p_refs):
        agg = agg_ref[0] + agg_ref[1]
        hn = jnp.dot(agg, wagg_ref[...], preferred_element_type=jnp.float32, precision=lax.Precision.HIGHEST)
        hn = hn + jnp.dot(h_ref[...], wself_ref[...], preferred_element_type=jnp.float32, precision=lax.Precision.HIGHEST)
        hn = jnp.maximum(hn, 0.0)
        hn_ref[...] = hn
        hw_ref[...] = hn * whint_ref[...]
        if with_p:
            p_refs[0][...] = (
                jnp.dot(hn + xe_ref[...], wmsg_ref[...],
                        preferred_element_type=jnp.float32, precision=lax.Precision.HIGHEST) + b_ref[...])

    n_out = 3 if with_p else 2
    return pl.pallas_call(
        body,
        grid=(N // R,),
        in_specs=[agg_spec, row, row, full, full, full, vec, vec],
        out_specs=[row] * n_out,
        out_shape=[jax.ShapeDtypeStruct((N, H), jnp.float32)] * n_out,
    )


def _make_tc_out(N, H, R):
    row = pl.BlockSpec((R, H), lambda i: (i, 0))
    vec = pl.BlockSpec((1, H), lambda i: (0, 0))
    out_spec = pl.BlockSpec((R, 1), lambda i: (i, 0))

    def body(h_ref, wout_ref, o_ref):
        o_ref[...] = jnp.sum(h_ref[...] * wout_ref[...], axis=1, keepdims=True)

    return pl.pallas_call(
        body,
        grid=(N // R,),
        in_specs=[row, vec],
        out_specs=out_spec,
        out_shape=jax.ShapeDtypeStruct((N, 1), jnp.float32),
    )


def kernel(x, edge_index, trace_h, W_enc, W_msg, b_msg, w_eh, W_agg, W_self,
           w_hint, W_out):
    N, D = x.shape
    H = W_msg.shape[0]
    T, E = trace_h.shape
    R = 1000
    N_pad = -(-N // (NS * 128)) * (NS * 128)
    assert N % R == 0 and E % C == 0

    src = edge_index[0]
    dst = edge_index[1]
    b2 = b_msg.reshape(1, H)
    whint2 = w_hint.reshape(1, H)

    sc_msg = _make_sc_msg(N_pad, E, H)
    sc_hint = _make_sc_hint(N, E, H)
    tc_encode = _make_tc_encode(N, D, H, R)
    tc_update = _make_tc_update(N, H, R, True)
    tc_update_last = _make_tc_update(N, H, R, False)
    tc_out = _make_tc_out(N, H, R)

    xe, p = tc_encode(x, W_enc, W_msg, b2)
    h = jnp.zeros((N, H), dtype=x.dtype)
    preds = []
    for t in range(T):
        agg = sc_msg(p, src, dst, trace_h[t], w_eh)
        if t + 1 < T:
            h, hw, p = tc_update(agg, h, xe, W_agg, W_self, W_msg, b2, whint2)
        else:
            h, hw = tc_update_last(agg, h, xe, W_agg, W_self, W_msg, b2, whint2)
        preds.append(sc_hint(h, hw, src, dst))
    out = tc_out(h, W_out.reshape(1, H))
    return out, jnp.stack(preds, axis=0)
